# baseline (speedup 1.0000x reference)
import jax
import jax.numpy as jnp
from jax import lax
from jax.experimental import pallas as pl
from jax.experimental.pallas import tpu as pltpu
from jax.experimental.pallas import tpu_sc as plsc

N_NODES = 50000
N_EDGES = 800000
E_BLK = 6400
ED_P = 32

_info = plsc.get_sparse_core_info()
_NC, _NS = _info.num_cores, _info.num_subcores
NW = _NC * _NS
NP = 51200
BPW = NP // NW
CH = 400
N_BLK = 2048


def _edge_body(idx_ref, tbl_ref, out_ref):
    idx = idx_ref[0, 0, :]
    tbl = jnp.maximum(tbl_ref[...], 0.0)
    oh = (lax.broadcasted_iota(jnp.int32, (E_BLK, ED_P), 1)
          == idx[:, None]).astype(jnp.float32)
    out_ref[...] = jnp.dot(oh, tbl, preferred_element_type=jnp.float32)


def _sc_gather_body(cat_hbm, rot_hbm, wcat_hbm, wor_hbm, gcat_hbm, gor_hbm,
                    idxc, idxo, bufc, bufo, semc, semo):
    wid = lax.axis_index("s") * _NC + lax.axis_index("c")
    base = wid * BPW
    for k in range(BPW // CH):
        off = base + k * CH
        pltpu.sync_copy(cat_hbm.at[pl.ds(off, CH)], idxc)
        pltpu.sync_copy(rot_hbm.at[pl.ds(off, CH)], idxo)
        cpc = pltpu.async_copy(wcat_hbm.at[idxc], bufc, semc)
        cpo = pltpu.async_copy(wor_hbm.at[idxo], bufo, semo)
        cpc.wait()
        cpo.wait()
        pltpu.sync_copy(bufc, gcat_hbm.at[pl.ds(off, CH)])
        pltpu.sync_copy(bufo, gor_hbm.at[pl.ds(off, CH)])


def _node_body(gcat_ref, gor_ref, dl_ref, Wn1_ref, Wn2_ref, Wn3_ref, Wn4_ref,
               Ws_ref, Wt_ref, bs_ref, bt_ref, bn_ref, out_ref):
    nt = (((1,), (1,)), ((), ()))
    A_dim = lax.dot_general(Ws_ref[...], Wn3_ref[...],
                            (((0,), (1,)), ((), ())),
                            preferred_element_type=jnp.float32)
    A_loc = lax.dot_general(Wt_ref[...], Wn4_ref[...],
                            (((0,), (1,)), ((), ())),
                            preferred_element_type=jnp.float32)
    b_eff = (lax.dot_general(bs_ref[...], Wn3_ref[...], nt,
                             preferred_element_type=jnp.float32)
             + lax.dot_general(bt_ref[...], Wn4_ref[...], nt,
                               preferred_element_type=jnp.float32)
             + bn_ref[...])
    A = jnp.concatenate([A_dim, A_loc, b_eff,
                         jnp.zeros((1, 64), jnp.float32)], axis=0)
    acc = lax.dot_general(gcat_ref[...], Wn1_ref[...], nt,
                          preferred_element_type=jnp.float32)
    acc += lax.dot_general(gor_ref[...], Wn2_ref[...], nt,
                           preferred_element_type=jnp.float32)
    acc += jnp.dot(dl_ref[...], A, preferred_element_type=jnp.float32)
    out_ref[...] = jnp.maximum(acc, 0.0)


def kernel(category, rotation_z, location, dimension, edge_feat,
           W_cat, W_orient, W_size, b_size, W_trans, b_trans,
           W_node, b_node, W_edge):
    f32 = jnp.float32
    eidx = edge_feat.astype(jnp.int32).reshape(N_EDGES // E_BLK, 1, E_BLK)
    We_p = jnp.pad(W_edge, ((0, ED_P - W_edge.shape[0]), (0, 0)))

    cat_p = jnp.pad(category.astype(jnp.int32), (0, NP - N_NODES))
    rot_p = jnp.pad(rotation_z.astype(jnp.int32), (0, NP - N_NODES))
    dl = jnp.concatenate(
        [dimension.astype(f32), location.astype(f32),
         jnp.ones((N_NODES, 1), f32), jnp.zeros((N_NODES, 1), f32)], axis=1)
    dl = jnp.pad(dl, ((0, NP - N_NODES), (0, 0)))
    Wn1 = W_node[:, 0:64]
    Wn2 = W_node[:, 64:96]
    Wn3 = W_node[:, 96:112]
    Wn4 = W_node[:, 112:128]
    bs2 = b_size.reshape(1, 16)
    bt2 = b_trans.reshape(1, 16)
    bn2 = b_node.reshape(1, 64)

    m_edge = pl.pallas_call(
        _edge_body,
        grid=(N_EDGES // E_BLK,),
        in_specs=[
            pl.BlockSpec((1, 1, E_BLK), lambda i: (i, 0, 0)),
            pl.BlockSpec((ED_P, 64), lambda i: (0, 0)),
        ],
        out_specs=pl.BlockSpec((E_BLK, 64), lambda i: (i, 0)),
        out_shape=jax.ShapeDtypeStruct((N_EDGES, 64), f32),
    )(eidx, We_p)

    sc_gather = pl.kernel(
        _sc_gather_body,
        out_type=(jax.ShapeDtypeStruct((NP, 64), f32),
                  jax.ShapeDtypeStruct((NP, 32), f32)),
        mesh=plsc.VectorSubcoreMesh(core_axis_name="c", subcore_axis_name="s"),
        scratch_types=[
            pltpu.VMEM((CH,), jnp.int32),
            pltpu.VMEM((CH,), jnp.int32),
            pltpu.VMEM((CH, 64), f32),
            pltpu.VMEM((CH, 32), f32),
            pltpu.SemaphoreType.DMA,
            pltpu.SemaphoreType.DMA,
        ],
        compiler_params=pltpu.CompilerParams(use_tc_tiling_on_sc=False),
    )
    gcat, gor = sc_gather(cat_p, rot_p, W_cat, W_orient)

    full = lambda shape: pl.BlockSpec(shape, lambda i: tuple(0 for _ in shape))
    m_node = pl.pallas_call(
        _node_body,
        grid=(NP // N_BLK,),
        in_specs=[
            pl.BlockSpec((N_BLK, 64), lambda i: (i, 0)),
            pl.BlockSpec((N_BLK, 32), lambda i: (i, 0)),
            pl.BlockSpec((N_BLK, 8), lambda i: (i, 0)),
            full((64, 64)), full((64, 32)), full((64, 16)), full((64, 16)),
            full((16, 3)), full((16, 3)),
            full((1, 16)), full((1, 16)), full((1, 64)),
        ],
        out_specs=pl.BlockSpec((N_BLK, 64), lambda i: (i, 0)),
        out_shape=jax.ShapeDtypeStruct((NP, 64), f32),
    )(gcat, gor, dl, Wn1, Wn2, Wn3, Wn4, W_size, W_trans, bs2, bt2, bn2)

    return (m_node[:N_NODES], m_edge)


# speedup vs baseline: 2.6252x; 2.6252x over previous
import jax
import jax.numpy as jnp
from jax import lax
from jax.experimental import pallas as pl
from jax.experimental.pallas import tpu as pltpu
from jax.experimental.pallas import tpu_sc as plsc

N_NODES = 50000
N_EDGES = 800000
E_BLK = 6400
ED_P = 32

_NC, _NS = 2, 16
NW = _NC * _NS
NP = 51200
BPW = NP // NW
CH = 400
N_BLK = 2048


def _edge_body(idx_ref, tbl_ref, out_ref):
    idx = idx_ref[0, 0, :]
    tbl = jnp.maximum(tbl_ref[...], 0.0)
    oh = (lax.broadcasted_iota(jnp.int32, (ED_P, E_BLK), 0)
          == idx[None, :]).astype(jnp.float32)
    out_ref[...] = jnp.dot(tbl, oh, preferred_element_type=jnp.float32)


def _sc_gather_body(cat_hbm, rot_hbm, wcat_hbm, wor_hbm, gcat_hbm, gor_hbm,
                    idxc, idxo, bufc, bufo, semc, semo):
    wid = lax.axis_index("s") * _NC + lax.axis_index("c")
    base = wid * BPW
    for k in range(BPW // CH):
        off = base + k * CH
        pltpu.sync_copy(cat_hbm.at[pl.ds(off, CH)], idxc)
        pltpu.sync_copy(rot_hbm.at[pl.ds(off, CH)], idxo)
        cpc = pltpu.async_copy(wcat_hbm.at[idxc], bufc, semc)
        cpo = pltpu.async_copy(wor_hbm.at[idxo], bufo, semo)
        cpc.wait()
        cpo.wait()
        pltpu.sync_copy(bufc, gcat_hbm.at[pl.ds(off, CH)])
        pltpu.sync_copy(bufo, gor_hbm.at[pl.ds(off, CH)])


def _node_body(gcat_ref, gor_ref, dlt_ref, Wn1_ref, Wn2_ref, Wn3_ref,
               Wn4_ref, Ws_ref, Wt_ref, bs_ref, bt_ref, bn_ref, out_ref):
    nt = (((1,), (1,)), ((), ()))
    A_dim = lax.dot_general(Ws_ref[...], Wn3_ref[...],
                            (((0,), (1,)), ((), ())),
                            preferred_element_type=jnp.float32)
    A_loc = lax.dot_general(Wt_ref[...], Wn4_ref[...],
                            (((0,), (1,)), ((), ())),
                            preferred_element_type=jnp.float32)
    b_eff = (lax.dot_general(bs_ref[...], Wn3_ref[...], nt,
                             preferred_element_type=jnp.float32)
             + lax.dot_general(bt_ref[...], Wn4_ref[...], nt,
                               preferred_element_type=jnp.float32)
             + bn_ref[...])
    A = jnp.concatenate([A_dim, A_loc, b_eff,
                         jnp.zeros((1, 64), jnp.float32)], axis=0)
    acc = lax.dot_general(Wn1_ref[...], gcat_ref[...], nt,
                          preferred_element_type=jnp.float32)
    acc += lax.dot_general(Wn2_ref[...], gor_ref[...], nt,
                           preferred_element_type=jnp.float32)
    acc += lax.dot_general(A, dlt_ref[...], (((0,), (0,)), ((), ())),
                           preferred_element_type=jnp.float32)
    out_ref[...] = jnp.maximum(acc, 0.0)


def kernel(category, rotation_z, location, dimension, edge_feat,
           W_cat, W_orient, W_size, b_size, W_trans, b_trans,
           W_node, b_node, W_edge):
    f32 = jnp.float32
    eidx = edge_feat.astype(jnp.int32).reshape(N_EDGES // E_BLK, 1, E_BLK)
    WeT_p = jnp.pad(W_edge.T, ((0, 0), (0, ED_P - W_edge.shape[0])))

    cat_p = jnp.pad(category.astype(jnp.int32), (0, NP - N_NODES))
    rot_p = jnp.pad(rotation_z.astype(jnp.int32), (0, NP - N_NODES))
    dlt = jnp.concatenate(
        [dimension.T.astype(f32), location.T.astype(f32),
         jnp.ones((1, N_NODES), f32), jnp.zeros((1, N_NODES), f32)], axis=0)
    dlt = jnp.pad(dlt, ((0, 0), (0, NP - N_NODES)))
    Wc_p = jnp.pad(W_cat, ((0, 0), (0, 64)))
    Wo_p = jnp.pad(W_orient, ((0, 0), (0, 96)))
    Wn1 = jnp.pad(W_node[:, 0:64], ((0, 0), (0, 64)))
    Wn2 = jnp.pad(W_node[:, 64:96], ((0, 0), (0, 96)))
    Wn3 = W_node[:, 96:112]
    Wn4 = W_node[:, 112:128]
    bs2 = b_size.reshape(1, 16)
    bt2 = b_trans.reshape(1, 16)
    bn2 = b_node.reshape(1, 64)

    m_edge_t = pl.pallas_call(
        _edge_body,
        grid=(N_EDGES // E_BLK,),
        in_specs=[
            pl.BlockSpec((1, 1, E_BLK), lambda i: (i, 0, 0)),
            pl.BlockSpec((64, ED_P), lambda i: (0, 0)),
        ],
        out_specs=pl.BlockSpec((64, E_BLK), lambda i: (0, i)),
        out_shape=jax.ShapeDtypeStruct((64, N_EDGES), f32),
    )(eidx, WeT_p)

    sc_gather = pl.kernel(
        _sc_gather_body,
        out_type=(jax.ShapeDtypeStruct((NP, 128), f32),
                  jax.ShapeDtypeStruct((NP, 128), f32)),
        mesh=plsc.VectorSubcoreMesh(core_axis_name="c", subcore_axis_name="s"),
        scratch_types=[
            pltpu.VMEM((CH,), jnp.int32),
            pltpu.VMEM((CH,), jnp.int32),
            pltpu.VMEM((CH, 128), f32),
            pltpu.VMEM((CH, 128), f32),
            pltpu.SemaphoreType.DMA,
            pltpu.SemaphoreType.DMA,
        ],
        compiler_params=pltpu.CompilerParams(use_tc_tiling_on_sc=True),
    )
    gcat, gor = sc_gather(cat_p, rot_p, Wc_p, Wo_p)

    full = lambda shape: pl.BlockSpec(shape, lambda i: tuple(0 for _ in shape))
    m_node_t = pl.pallas_call(
        _node_body,
        grid=(NP // N_BLK,),
        in_specs=[
            pl.BlockSpec((N_BLK, 128), lambda i: (i, 0)),
            pl.BlockSpec((N_BLK, 128), lambda i: (i, 0)),
            pl.BlockSpec((8, N_BLK), lambda i: (0, i)),
            full((64, 128)), full((64, 128)), full((64, 16)), full((64, 16)),
            full((16, 3)), full((16, 3)),
            full((1, 16)), full((1, 16)), full((1, 64)),
        ],
        out_specs=pl.BlockSpec((64, N_BLK), lambda i: (0, i)),
        out_shape=jax.ShapeDtypeStruct((64, NP), f32),
    )(gcat, gor, dlt, Wn1, Wn2, Wn3, Wn4, W_size, W_trans, bs2, bt2, bn2)

    return (m_node_t[:, :N_NODES].T, m_edge_t.T)


# speedup vs baseline: 2.7078x; 1.0315x over previous
import jax
import jax.numpy as jnp
from jax import lax
from jax.experimental import pallas as pl
from jax.experimental.pallas import tpu as pltpu
from jax.experimental.pallas import tpu_sc as plsc

N_NODES = 50000
N_EDGES = 800000
E_BLK = 16000
ED_P = 32

_NC, _NS = 2, 16
NW = _NC * _NS
NP = 51200
BPW = NP // NW
CH = 200
N_BLK = 2048


def _edge_body(idx_ref, tbl_ref, out_ref):
    idx = idx_ref[0, 0, :]
    tbl = jnp.maximum(tbl_ref[...], 0.0)
    oh = (lax.broadcasted_iota(jnp.int32, (ED_P, E_BLK), 0)
          == idx[None, :]).astype(jnp.float32)
    out_ref[...] = jnp.dot(tbl, oh, preferred_element_type=jnp.float32)


def _sc_gather_body(cat_hbm, rot_hbm, wcat_hbm, wor_hbm, gcat_hbm, gor_hbm,
                    idxc, idxo, bufc0, bufc1, bufo0, bufo1,
                    sgc0, sgc1, sgo0, sgo1, swc0, swc1, swo0, swo1):
    wid = lax.axis_index("s") * _NC + lax.axis_index("c")
    base = wid * BPW
    nch = BPW // CH
    bufc = (bufc0, bufc1)
    bufo = (bufo0, bufo1)
    sg = ((sgc0, sgo0), (sgc1, sgo1))
    sw = ((swc0, swo0), (swc1, swo1))
    pltpu.sync_copy(cat_hbm.at[pl.ds(base, BPW)], idxc)
    pltpu.sync_copy(rot_hbm.at[pl.ds(base, BPW)], idxo)

    def start_gather(k, s):
        return (pltpu.async_copy(wcat_hbm.at[idxc.at[pl.ds(k * CH, CH)]],
                                 bufc[s], sg[s][0]),
                pltpu.async_copy(wor_hbm.at[idxo.at[pl.ds(k * CH, CH)]],
                                 bufo[s], sg[s][1]))

    g = [None, None]
    w = [None, None]
    g[0] = start_gather(0, 0)
    for k in range(nch):
        s = k & 1
        t = 1 - s
        g[s][0].wait()
        g[s][1].wait()
        if k + 1 < nch:
            if k >= 1:
                w[t][0].wait()
                w[t][1].wait()
            g[t] = start_gather(k + 1, t)
        off = base + k * CH
        w[s] = (pltpu.async_copy(bufc[s], gcat_hbm.at[pl.ds(off, CH)],
                                 sw[s][0]),
                pltpu.async_copy(bufo[s], gor_hbm.at[pl.ds(off, CH)],
                                 sw[s][1]))
    for s in range(2):
        w[s][0].wait()
        w[s][1].wait()


def _node_body(gcat_ref, gor_ref, dlt_ref, Wn1_ref, Wn2_ref, Wn3_ref,
               Wn4_ref, Ws_ref, Wt_ref, bs_ref, bt_ref, bn_ref, out_ref):
    nt = (((1,), (1,)), ((), ()))
    A_dim = lax.dot_general(Ws_ref[...], Wn3_ref[...],
                            (((0,), (1,)), ((), ())),
                            preferred_element_type=jnp.float32)
    A_loc = lax.dot_general(Wt_ref[...], Wn4_ref[...],
                            (((0,), (1,)), ((), ())),
                            preferred_element_type=jnp.float32)
    b_eff = (lax.dot_general(bs_ref[...], Wn3_ref[...], nt,
                             preferred_element_type=jnp.float32)
             + lax.dot_general(bt_ref[...], Wn4_ref[...], nt,
                               preferred_element_type=jnp.float32)
             + bn_ref[...])
    A = jnp.concatenate([A_dim, A_loc, b_eff,
                         jnp.zeros((1, 64), jnp.float32)], axis=0)
    acc = lax.dot_general(Wn1_ref[...], gcat_ref[...], nt,
                          preferred_element_type=jnp.float32)
    acc += lax.dot_general(Wn2_ref[...], gor_ref[...], nt,
                           preferred_element_type=jnp.float32)
    acc += lax.dot_general(A, dlt_ref[...], (((0,), (0,)), ((), ())),
                           preferred_element_type=jnp.float32)
    out_ref[...] = jnp.maximum(acc, 0.0)


def kernel(category, rotation_z, location, dimension, edge_feat,
           W_cat, W_orient, W_size, b_size, W_trans, b_trans,
           W_node, b_node, W_edge):
    f32 = jnp.float32
    eidx = edge_feat.astype(jnp.int32).reshape(N_EDGES // E_BLK, 1, E_BLK)
    WeT_p = jnp.pad(W_edge.T, ((0, 0), (0, ED_P - W_edge.shape[0])))

    cat_p = jnp.pad(category.astype(jnp.int32), (0, NP - N_NODES))
    rot_p = jnp.pad(rotation_z.astype(jnp.int32), (0, NP - N_NODES))
    dlt = jnp.concatenate(
        [dimension.T.astype(f32), location.T.astype(f32),
         jnp.ones((1, N_NODES), f32), jnp.zeros((1, N_NODES), f32)], axis=0)
    dlt = jnp.pad(dlt, ((0, 0), (0, NP - N_NODES)))
    Wc_p = jnp.pad(W_cat, ((0, 0), (0, 64)))
    Wo_p = jnp.pad(W_orient, ((0, 0), (0, 96)))
    Wn1 = jnp.pad(W_node[:, 0:64], ((0, 0), (0, 64)))
    Wn2 = jnp.pad(W_node[:, 64:96], ((0, 0), (0, 96)))
    Wn3 = W_node[:, 96:112]
    Wn4 = W_node[:, 112:128]
    bs2 = b_size.reshape(1, 16)
    bt2 = b_trans.reshape(1, 16)
    bn2 = b_node.reshape(1, 64)

    m_edge_t = pl.pallas_call(
        _edge_body,
        grid=(N_EDGES // E_BLK,),
        in_specs=[
            pl.BlockSpec((1, 1, E_BLK), lambda i: (i, 0, 0)),
            pl.BlockSpec((64, ED_P), lambda i: (0, 0)),
        ],
        out_specs=pl.BlockSpec((64, E_BLK), lambda i: (0, i)),
        out_shape=jax.ShapeDtypeStruct((64, N_EDGES), f32),
    )(eidx, WeT_p)

    sc_gather = pl.kernel(
        _sc_gather_body,
        out_type=(jax.ShapeDtypeStruct((NP, 128), f32),
                  jax.ShapeDtypeStruct((NP, 128), f32)),
        mesh=plsc.VectorSubcoreMesh(core_axis_name="c", subcore_axis_name="s"),
        scratch_types=(
            [pltpu.VMEM((BPW,), jnp.int32)] * 2
            + [pltpu.VMEM((CH, 128), f32)] * 4
            + [pltpu.SemaphoreType.DMA] * 8
        ),
        compiler_params=pltpu.CompilerParams(use_tc_tiling_on_sc=True),
    )
    gcat, gor = sc_gather(cat_p, rot_p, Wc_p, Wo_p)

    full = lambda shape: pl.BlockSpec(shape, lambda i: tuple(0 for _ in shape))
    m_node_t = pl.pallas_call(
        _node_body,
        grid=(NP // N_BLK,),
        in_specs=[
            pl.BlockSpec((N_BLK, 128), lambda i: (i, 0)),
            pl.BlockSpec((N_BLK, 128), lambda i: (i, 0)),
            pl.BlockSpec((8, N_BLK), lambda i: (0, i)),
            full((64, 128)), full((64, 128)), full((64, 16)), full((64, 16)),
            full((16, 3)), full((16, 3)),
            full((1, 16)), full((1, 16)), full((1, 64)),
        ],
        out_specs=pl.BlockSpec((64, N_BLK), lambda i: (0, i)),
        out_shape=jax.ShapeDtypeStruct((64, NP), f32),
    )(gcat, gor, dlt, Wn1, Wn2, Wn3, Wn4, W_size, W_trans, bs2, bt2, bn2)

    return (m_node_t[:, :N_NODES].T, m_edge_t.T)


# speedup vs baseline: 2.8791x; 1.0633x over previous
import jax
import jax.numpy as jnp
from jax import lax
from jax.experimental import pallas as pl
from jax.experimental.pallas import tpu as pltpu
from jax.experimental.pallas import tpu_sc as plsc

N_NODES = 50000
N_EDGES = 800000
E_BLK = 16000
ED_P = 32
N_OR = 360

_NC, _NS = 2, 16
NW = _NC * _NS
NP = 51200
BPW = NP // NW
CH = 400
N_BLK = 2048


def _edge_body(idx_ref, tbl_ref, out_ref):
    idx = idx_ref[0, 0, :]
    tbl = jnp.maximum(tbl_ref[...], 0.0)
    oh = (lax.broadcasted_iota(jnp.int32, (ED_P, E_BLK), 0)
          == idx[None, :]).astype(jnp.float32)
    out_ref[...] = jnp.dot(tbl, oh, preferred_element_type=jnp.float32)


def _sc_gather_body(cat_hbm, wcat_hbm, gcat_hbm,
                    idxc, bufc0, bufc1, sg0, sg1, sw0, sw1):
    wid = lax.axis_index("s") * _NC + lax.axis_index("c")
    base = wid * BPW
    nch = BPW // CH
    bufc = (bufc0, bufc1)
    sg = (sg0, sg1)
    sw = (sw0, sw1)
    pltpu.sync_copy(cat_hbm.at[pl.ds(base, BPW)], idxc)

    def start_gather(k, s):
        return pltpu.async_copy(wcat_hbm.at[idxc.at[pl.ds(k * CH, CH)]],
                                bufc[s], sg[s])

    g = [None, None]
    w = [None, None]
    g[0] = start_gather(0, 0)
    for k in range(nch):
        s = k & 1
        t = 1 - s
        g[s].wait()
        if k + 1 < nch:
            if k >= 1:
                w[t].wait()
            g[t] = start_gather(k + 1, t)
        off = base + k * CH
        w[s] = pltpu.async_copy(bufc[s], gcat_hbm.at[pl.ds(off, CH)], sw[s])
    for s in range(2):
        if w[s] is not None:
            w[s].wait()


def _node_body(gcat_ref, rot_ref, dlt_ref, Wor_ref, Wn1_ref, Wn2_ref,
               Wn3_ref, Wn4_ref, Ws_ref, Wt_ref, bs_ref, bt_ref, bn_ref,
               out_ref):
    nt = (((1,), (1,)), ((), ()))
    A_dim = lax.dot_general(Ws_ref[...], Wn3_ref[...],
                            (((0,), (1,)), ((), ())),
                            preferred_element_type=jnp.float32)
    A_loc = lax.dot_general(Wt_ref[...], Wn4_ref[...],
                            (((0,), (1,)), ((), ())),
                            preferred_element_type=jnp.float32)
    b_eff = (lax.dot_general(bs_ref[...], Wn3_ref[...], nt,
                             preferred_element_type=jnp.float32)
             + lax.dot_general(bt_ref[...], Wn4_ref[...], nt,
                               preferred_element_type=jnp.float32)
             + bn_ref[...])
    A = jnp.concatenate([A_dim, A_loc, b_eff,
                         jnp.zeros((1, 64), jnp.float32)], axis=0)
    T_or = lax.dot_general(Wn2_ref[...], Wor_ref[...], nt,
                           preferred_element_type=jnp.float32)
    rot = rot_ref[0, :]
    oho = (lax.broadcasted_iota(jnp.int32, (N_OR, N_BLK), 0)
           == rot[None, :]).astype(jnp.float32)
    acc = lax.dot_general(Wn1_ref[...], gcat_ref[...], nt,
                          preferred_element_type=jnp.float32)
    acc += jnp.dot(T_or, oho, preferred_element_type=jnp.float32)
    acc += lax.dot_general(A, dlt_ref[...], (((0,), (0,)), ((), ())),
                           preferred_element_type=jnp.float32)
    out_ref[...] = jnp.maximum(acc, 0.0)


def kernel(category, rotation_z, location, dimension, edge_feat,
           W_cat, W_orient, W_size, b_size, W_trans, b_trans,
           W_node, b_node, W_edge):
    f32 = jnp.float32
    eidx = edge_feat.astype(jnp.int32).reshape(N_EDGES // E_BLK, 1, E_BLK)
    WeT_p = jnp.pad(W_edge.T, ((0, 0), (0, ED_P - W_edge.shape[0])))

    cat_p = jnp.pad(category.astype(jnp.int32), (0, NP - N_NODES))
    rot2 = jnp.pad(rotation_z.astype(jnp.int32), (0, NP - N_NODES)).reshape(1, NP)
    dlt = jnp.concatenate(
        [dimension.T.astype(f32), location.T.astype(f32),
         jnp.ones((1, N_NODES), f32), jnp.zeros((1, N_NODES), f32)], axis=0)
    dlt = jnp.pad(dlt, ((0, 0), (0, NP - N_NODES)))
    Wc_p = jnp.pad(W_cat, ((0, 0), (0, 64)))
    Wn1 = jnp.pad(W_node[:, 0:64], ((0, 0), (0, 64)))
    Wn2 = W_node[:, 64:96]
    Wn3 = W_node[:, 96:112]
    Wn4 = W_node[:, 112:128]
    bs2 = b_size.reshape(1, 16)
    bt2 = b_trans.reshape(1, 16)
    bn2 = b_node.reshape(1, 64)

    m_edge_t = pl.pallas_call(
        _edge_body,
        grid=(N_EDGES // E_BLK,),
        in_specs=[
            pl.BlockSpec((1, 1, E_BLK), lambda i: (i, 0, 0)),
            pl.BlockSpec((64, ED_P), lambda i: (0, 0)),
        ],
        out_specs=pl.BlockSpec((64, E_BLK), lambda i: (0, i)),
        out_shape=jax.ShapeDtypeStruct((64, N_EDGES), f32),
    )(eidx, WeT_p)

    sc_gather = pl.kernel(
        _sc_gather_body,
        out_type=jax.ShapeDtypeStruct((NP, 128), f32),
        mesh=plsc.VectorSubcoreMesh(core_axis_name="c", subcore_axis_name="s"),
        scratch_types=(
            [pltpu.VMEM((BPW,), jnp.int32)]
            + [pltpu.VMEM((CH, 128), f32)] * 2
            + [pltpu.SemaphoreType.DMA] * 4
        ),
        compiler_params=pltpu.CompilerParams(use_tc_tiling_on_sc=True),
    )
    gcat = sc_gather(cat_p, Wc_p)

    full = lambda shape: pl.BlockSpec(shape, lambda i: tuple(0 for _ in shape))
    m_node_t = pl.pallas_call(
        _node_body,
        grid=(NP // N_BLK,),
        in_specs=[
            pl.BlockSpec((N_BLK, 128), lambda i: (i, 0)),
            pl.BlockSpec((1, N_BLK), lambda i: (0, i)),
            pl.BlockSpec((8, N_BLK), lambda i: (0, i)),
            full((N_OR, 32)),
            full((64, 128)), full((64, 32)), full((64, 16)), full((64, 16)),
            full((16, 3)), full((16, 3)),
            full((1, 16)), full((1, 16)), full((1, 64)),
        ],
        out_specs=pl.BlockSpec((64, N_BLK), lambda i: (0, i)),
        out_shape=jax.ShapeDtypeStruct((64, NP), f32),
    )(gcat, rot2, dlt, W_orient, Wn1, Wn2, Wn3, Wn4,
      W_size, W_trans, bs2, bt2, bn2)

    return (m_node_t[:, :N_NODES].T, m_edge_t.T)


# speedup vs baseline: 3.3335x; 1.1578x over previous
import jax
import jax.numpy as jnp
from jax import lax
from jax.experimental import pallas as pl
from jax.experimental.pallas import tpu as pltpu
from jax.experimental.pallas import tpu_sc as plsc

N_NODES = 50000
N_EDGES = 800000
E_BLK = 16000
ED_P = 32
N_OR = 360

_NC, _NS = 2, 16
NW = _NC * _NS
NP = 51200
BPW = NP // NW
CH = 400
N_BLK = 2048


def _edge_body(idx_ref, tbl_ref, out_ref):
    idx = idx_ref[0, 0, :]
    tbl = jnp.maximum(tbl_ref[...], 0.0)
    oh = (lax.broadcasted_iota(jnp.int32, (ED_P, E_BLK), 0)
          == idx[None, :]).astype(jnp.float32)
    out_ref[...] = jnp.dot(tbl, oh, preferred_element_type=jnp.float32)


def _sc_gather_body(cat_hbm, wcat_hbm, gcat_hbm,
                    idxc, bufc0, bufc1, sg0, sg1, sw0, sw1):
    wid = lax.axis_index("s") * _NC + lax.axis_index("c")
    base = wid * BPW
    nch = BPW // CH
    bufc = (bufc0, bufc1)
    sg = (sg0, sg1)
    sw = (sw0, sw1)
    pltpu.sync_copy(cat_hbm.at[pl.ds(base, BPW)], idxc)

    def start_gather(k, s):
        return pltpu.async_copy(wcat_hbm.at[idxc.at[pl.ds(k * CH, CH)]],
                                bufc[s], sg[s])

    g = [None, None]
    w = [None, None]
    g[0] = start_gather(0, 0)
    for k in range(nch):
        s = k & 1
        t = 1 - s
        g[s].wait()
        if k + 1 < nch:
            if k >= 1:
                w[t].wait()
            g[t] = start_gather(k + 1, t)
        off = base + k * CH
        w[s] = pltpu.async_copy(bufc[s], gcat_hbm.at[pl.ds(off, CH)], sw[s])
    for s in range(2):
        if w[s] is not None:
            w[s].wait()


def _node_body(gcat_ref, rot_ref, dlt_ref, Wor_ref, Wn1_ref, Wn2_ref,
               Wn3_ref, Wn4_ref, Ws_ref, Wt_ref, bs_ref, bt_ref, bn_ref,
               out_ref):
    nt = (((1,), (1,)), ((), ()))
    A_dim = lax.dot_general(Ws_ref[...], Wn3_ref[...],
                            (((0,), (1,)), ((), ())),
                            preferred_element_type=jnp.float32)
    A_loc = lax.dot_general(Wt_ref[...], Wn4_ref[...],
                            (((0,), (1,)), ((), ())),
                            preferred_element_type=jnp.float32)
    b_eff = (lax.dot_general(bs_ref[...], Wn3_ref[...], nt,
                             preferred_element_type=jnp.float32)
             + lax.dot_general(bt_ref[...], Wn4_ref[...], nt,
                               preferred_element_type=jnp.float32)
             + bn_ref[...])
    A = jnp.concatenate([A_dim, A_loc, b_eff,
                         jnp.zeros((1, 64), jnp.float32)], axis=0)
    T_or = lax.dot_general(Wn2_ref[...], Wor_ref[...], nt,
                           preferred_element_type=jnp.float32)
    rot = rot_ref[0, :]
    oho = (lax.broadcasted_iota(jnp.int32, (N_OR, N_BLK), 0)
           == rot[None, :]).astype(jnp.float32)
    acc = lax.dot_general(Wn1_ref[...], gcat_ref[...], nt,
                          preferred_element_type=jnp.float32)
    acc += jnp.dot(T_or, oho, preferred_element_type=jnp.float32)
    acc += lax.dot_general(A, dlt_ref[...], (((0,), (0,)), ((), ())),
                           preferred_element_type=jnp.float32)
    out_ref[...] = jnp.maximum(acc, 0.0)


def kernel(category, rotation_z, location, dimension, edge_feat,
           W_cat, W_orient, W_size, b_size, W_trans, b_trans,
           W_node, b_node, W_edge):
    f32 = jnp.float32
    eidx = edge_feat.astype(jnp.int32).reshape(N_EDGES // E_BLK, 1, E_BLK)
    WeT_p = jnp.pad(W_edge.T, ((0, 0), (0, ED_P - W_edge.shape[0])))

    cat_p = jnp.pad(category.astype(jnp.int32), (0, NP - N_NODES))
    rot2 = jnp.pad(rotation_z.astype(jnp.int32), (0, NP - N_NODES)).reshape(1, NP)
    dlt = jnp.concatenate(
        [dimension.T.astype(f32), location.T.astype(f32),
         jnp.ones((1, N_NODES), f32), jnp.zeros((1, N_NODES), f32)], axis=0)
    dlt = jnp.pad(dlt, ((0, 0), (0, NP - N_NODES)))
    Wc_p = W_cat
    Wn1 = W_node[:, 0:64]
    Wn2 = W_node[:, 64:96]
    Wn3 = W_node[:, 96:112]
    Wn4 = W_node[:, 112:128]
    bs2 = b_size.reshape(1, 16)
    bt2 = b_trans.reshape(1, 16)
    bn2 = b_node.reshape(1, 64)

    m_edge_t = pl.pallas_call(
        _edge_body,
        grid=(N_EDGES // E_BLK,),
        in_specs=[
            pl.BlockSpec((1, 1, E_BLK), lambda i: (i, 0, 0)),
            pl.BlockSpec((64, ED_P), lambda i: (0, 0)),
        ],
        out_specs=pl.BlockSpec((64, E_BLK), lambda i: (0, i)),
        out_shape=jax.ShapeDtypeStruct((64, N_EDGES), f32),
    )(eidx, WeT_p)

    sc_gather = pl.kernel(
        _sc_gather_body,
        out_type=jax.ShapeDtypeStruct((NP, 64), f32),
        mesh=plsc.VectorSubcoreMesh(core_axis_name="c", subcore_axis_name="s"),
        scratch_types=(
            [pltpu.VMEM((BPW,), jnp.int32)]
            + [pltpu.VMEM((CH, 64), f32)] * 2
            + [pltpu.SemaphoreType.DMA] * 4
        ),
        compiler_params=pltpu.CompilerParams(use_tc_tiling_on_sc=False),
    )
    gcat = sc_gather(cat_p, Wc_p)

    full = lambda shape: pl.BlockSpec(shape, lambda i: tuple(0 for _ in shape))
    m_node_t = pl.pallas_call(
        _node_body,
        grid=(NP // N_BLK,),
        in_specs=[
            pl.BlockSpec((N_BLK, 64), lambda i: (i, 0)),
            pl.BlockSpec((1, N_BLK), lambda i: (0, i)),
            pl.BlockSpec((8, N_BLK), lambda i: (0, i)),
            full((N_OR, 32)),
            full((64, 64)), full((64, 32)), full((64, 16)), full((64, 16)),
            full((16, 3)), full((16, 3)),
            full((1, 16)), full((1, 16)), full((1, 64)),
        ],
        out_specs=pl.BlockSpec((64, N_BLK), lambda i: (0, i)),
        out_shape=jax.ShapeDtypeStruct((64, NP), f32),
    )(gcat, rot2, dlt, W_orient, Wn1, Wn2, Wn3, Wn4,
      W_size, W_trans, bs2, bt2, bn2)

    return (m_node_t[:, :N_NODES].T, m_edge_t.T)


# speedup vs baseline: 3.3550x; 1.0065x over previous
import jax
import jax.numpy as jnp
from jax import lax
from jax.experimental import pallas as pl
from jax.experimental.pallas import tpu as pltpu
from jax.experimental.pallas import tpu_sc as plsc

N_NODES = 50000
N_EDGES = 800000
E_BLK = 32000
ED_P = 32
N_OR = 360

_NC, _NS = 2, 16
NW = _NC * _NS
NP = 51200
BPW = NP // NW
CH = 400
N_BLK = 2048


def _edge_body(idx_ref, tbl_ref, out_ref):
    idx = idx_ref[0, 0, :]
    tbl = jnp.maximum(tbl_ref[...], 0.0)
    oh = (lax.broadcasted_iota(jnp.int32, (ED_P, E_BLK), 0)
          == idx[None, :]).astype(jnp.float32)
    out_ref[...] = jnp.dot(tbl, oh, preferred_element_type=jnp.float32)


def _sc_gather_body(cat_hbm, wcat_hbm, gcat_hbm,
                    idxc, bufc0, bufc1, sg0, sg1, sw0, sw1):
    wid = lax.axis_index("s") * _NC + lax.axis_index("c")
    base = wid * BPW
    nch = BPW // CH
    bufc = (bufc0, bufc1)
    sg = (sg0, sg1)
    sw = (sw0, sw1)
    pltpu.sync_copy(cat_hbm.at[pl.ds(base, BPW)], idxc)

    def start_gather(k, s):
        return pltpu.async_copy(wcat_hbm.at[idxc.at[pl.ds(k * CH, CH)]],
                                bufc[s], sg[s])

    g = [None, None]
    w = [None, None]
    g[0] = start_gather(0, 0)
    for k in range(nch):
        s = k & 1
        t = 1 - s
        g[s].wait()
        if k + 1 < nch:
            if k >= 1:
                w[t].wait()
            g[t] = start_gather(k + 1, t)
        off = base + k * CH
        w[s] = pltpu.async_copy(bufc[s], gcat_hbm.at[pl.ds(off, CH)], sw[s])
    for s in range(2):
        if w[s] is not None:
            w[s].wait()


def _node_body(gcat_ref, rot_ref, dlt_ref, Wor_ref, Wn1_ref, Wn2_ref,
               Wn3_ref, Wn4_ref, Ws_ref, Wt_ref, bs_ref, bt_ref, bn_ref,
               out_ref):
    nt = (((1,), (1,)), ((), ()))
    A_dim = lax.dot_general(Ws_ref[...], Wn3_ref[...],
                            (((0,), (1,)), ((), ())),
                            preferred_element_type=jnp.float32)
    A_loc = lax.dot_general(Wt_ref[...], Wn4_ref[...],
                            (((0,), (1,)), ((), ())),
                            preferred_element_type=jnp.float32)
    b_eff = (lax.dot_general(bs_ref[...], Wn3_ref[...], nt,
                             preferred_element_type=jnp.float32)
             + lax.dot_general(bt_ref[...], Wn4_ref[...], nt,
                               preferred_element_type=jnp.float32)
             + bn_ref[...])
    A = jnp.concatenate([A_dim, A_loc, b_eff,
                         jnp.zeros((1, 64), jnp.float32)], axis=0)
    T_or = lax.dot_general(Wn2_ref[...], Wor_ref[...], nt,
                           preferred_element_type=jnp.float32)
    rot = rot_ref[0, :]
    oho = (lax.broadcasted_iota(jnp.int32, (N_OR, N_BLK), 0)
           == rot[None, :]).astype(jnp.float32)
    acc = lax.dot_general(Wn1_ref[...], gcat_ref[...], nt,
                          preferred_element_type=jnp.float32)
    acc += jnp.dot(T_or, oho, preferred_element_type=jnp.float32)
    acc += lax.dot_general(A, dlt_ref[...], (((0,), (0,)), ((), ())),
                           preferred_element_type=jnp.float32)
    out_ref[...] = jnp.maximum(acc, 0.0)


def kernel(category, rotation_z, location, dimension, edge_feat,
           W_cat, W_orient, W_size, b_size, W_trans, b_trans,
           W_node, b_node, W_edge):
    f32 = jnp.float32
    eidx = edge_feat.astype(jnp.int32).reshape(N_EDGES // E_BLK, 1, E_BLK)
    WeT_p = jnp.pad(W_edge.T, ((0, 0), (0, ED_P - W_edge.shape[0])))

    cat_p = jnp.pad(category.astype(jnp.int32), (0, NP - N_NODES))
    rot2 = jnp.pad(rotation_z.astype(jnp.int32), (0, NP - N_NODES)).reshape(1, NP)
    dlt = jnp.concatenate(
        [dimension.T.astype(f32), location.T.astype(f32),
         jnp.ones((1, N_NODES), f32), jnp.zeros((1, N_NODES), f32)], axis=0)
    dlt = jnp.pad(dlt, ((0, 0), (0, NP - N_NODES)))
    Wc_p = W_cat
    Wn1 = W_node[:, 0:64]
    Wn2 = W_node[:, 64:96]
    Wn3 = W_node[:, 96:112]
    Wn4 = W_node[:, 112:128]
    bs2 = b_size.reshape(1, 16)
    bt2 = b_trans.reshape(1, 16)
    bn2 = b_node.reshape(1, 64)

    m_edge_t = pl.pallas_call(
        _edge_body,
        grid=(N_EDGES // E_BLK,),
        in_specs=[
            pl.BlockSpec((1, 1, E_BLK), lambda i: (i, 0, 0)),
            pl.BlockSpec((64, ED_P), lambda i: (0, 0)),
        ],
        out_specs=pl.BlockSpec((64, E_BLK), lambda i: (0, i)),
        out_shape=jax.ShapeDtypeStruct((64, N_EDGES), f32),
    )(eidx, WeT_p)

    sc_gather = pl.kernel(
        _sc_gather_body,
        out_type=jax.ShapeDtypeStruct((NP, 64), f32),
        mesh=plsc.VectorSubcoreMesh(core_axis_name="c", subcore_axis_name="s"),
        scratch_types=(
            [pltpu.VMEM((BPW,), jnp.int32)]
            + [pltpu.VMEM((CH, 64), f32)] * 2
            + [pltpu.SemaphoreType.DMA] * 4
        ),
        compiler_params=pltpu.CompilerParams(use_tc_tiling_on_sc=False),
    )
    gcat = sc_gather(cat_p, Wc_p)

    full = lambda shape: pl.BlockSpec(shape, lambda i: tuple(0 for _ in shape))
    m_node_t = pl.pallas_call(
        _node_body,
        grid=(NP // N_BLK,),
        in_specs=[
            pl.BlockSpec((N_BLK, 64), lambda i: (i, 0)),
            pl.BlockSpec((1, N_BLK), lambda i: (0, i)),
            pl.BlockSpec((8, N_BLK), lambda i: (0, i)),
            full((N_OR, 32)),
            full((64, 64)), full((64, 32)), full((64, 16)), full((64, 16)),
            full((16, 3)), full((16, 3)),
            full((1, 16)), full((1, 16)), full((1, 64)),
        ],
        out_specs=pl.BlockSpec((64, N_BLK), lambda i: (0, i)),
        out_shape=jax.ShapeDtypeStruct((64, NP), f32),
    )(gcat, rot2, dlt, W_orient, Wn1, Wn2, Wn3, Wn4,
      W_size, W_trans, bs2, bt2, bn2)

    return (m_node_t[:, :N_NODES].T, m_edge_t.T)


# speedup vs baseline: 3.4193x; 1.0192x over previous
import jax
import jax.numpy as jnp
from jax import lax
from jax.experimental import pallas as pl
from jax.experimental.pallas import tpu as pltpu
from jax.experimental.pallas import tpu_sc as plsc

N_NODES = 50000
N_EDGES = 800000
E_BLK = 32000
ED_P = 32
N_OR = 360

_NC, _NS = 2, 16
NW = _NC * _NS
NP = 51200
BPW = NP // NW
CH = 400
N_BLK = 2560


def _edge_body(idx_ref, tbl_ref, out_ref):
    idx = idx_ref[0, 0, :]
    tbl = jnp.maximum(tbl_ref[...], 0.0)
    oh = (lax.broadcasted_iota(jnp.int32, (ED_P, E_BLK), 0)
          == idx[None, :]).astype(jnp.float32)
    out_ref[...] = jnp.dot(tbl, oh, preferred_element_type=jnp.float32)


def _sc_gather_body(cat_hbm, wcat_hbm, gcat_hbm,
                    idxc, bufc0, bufc1, sg0, sg1, sw0, sw1):
    wid = lax.axis_index("s") * _NC + lax.axis_index("c")
    base = wid * BPW
    nch = BPW // CH
    bufc = (bufc0, bufc1)
    sg = (sg0, sg1)
    sw = (sw0, sw1)
    pltpu.sync_copy(cat_hbm.at[pl.ds(base, BPW)], idxc)

    def start_gather(k, s):
        return pltpu.async_copy(wcat_hbm.at[idxc.at[pl.ds(k * CH, CH)]],
                                bufc[s], sg[s])

    g = [None, None]
    w = [None, None]
    g[0] = start_gather(0, 0)
    for k in range(nch):
        s = k & 1
        t = 1 - s
        g[s].wait()
        if k + 1 < nch:
            if k >= 1:
                w[t].wait()
            g[t] = start_gather(k + 1, t)
        off = base + k * CH
        w[s] = pltpu.async_copy(bufc[s], gcat_hbm.at[pl.ds(off, CH)], sw[s])
    for s in range(2):
        if w[s] is not None:
            w[s].wait()


def _node_body(gcat_ref, rot_ref, dlt_ref, Wor_ref, Wn1_ref, Wn2_ref,
               Wn3_ref, Wn4_ref, Ws_ref, Wt_ref, bs_ref, bt_ref, bn_ref,
               out_ref):
    nt = (((1,), (1,)), ((), ()))
    A_dim = lax.dot_general(Ws_ref[...], Wn3_ref[...],
                            (((0,), (1,)), ((), ())),
                            preferred_element_type=jnp.float32)
    A_loc = lax.dot_general(Wt_ref[...], Wn4_ref[...],
                            (((0,), (1,)), ((), ())),
                            preferred_element_type=jnp.float32)
    b_eff = (lax.dot_general(bs_ref[...], Wn3_ref[...], nt,
                             preferred_element_type=jnp.float32)
             + lax.dot_general(bt_ref[...], Wn4_ref[...], nt,
                               preferred_element_type=jnp.float32)
             + bn_ref[...])
    A = jnp.concatenate([A_dim, A_loc, b_eff,
                         jnp.zeros((1, 64), jnp.float32)], axis=0)
    T_or = lax.dot_general(Wn2_ref[...], Wor_ref[...], nt,
                           preferred_element_type=jnp.float32)
    rot = rot_ref[0, :]
    oho = (lax.broadcasted_iota(jnp.int32, (N_OR, N_BLK), 0)
           == rot[None, :]).astype(jnp.float32)
    acc = lax.dot_general(Wn1_ref[...], gcat_ref[...], nt,
                          preferred_element_type=jnp.float32)
    acc += jnp.dot(T_or, oho, preferred_element_type=jnp.float32)
    acc += lax.dot_general(A, dlt_ref[...], (((0,), (0,)), ((), ())),
                           preferred_element_type=jnp.float32)
    out_ref[...] = jnp.maximum(acc, 0.0)


def kernel(category, rotation_z, location, dimension, edge_feat,
           W_cat, W_orient, W_size, b_size, W_trans, b_trans,
           W_node, b_node, W_edge):
    f32 = jnp.float32
    eidx = edge_feat.astype(jnp.int32).reshape(N_EDGES // E_BLK, 1, E_BLK)
    WeT_p = jnp.pad(W_edge.T, ((0, 0), (0, ED_P - W_edge.shape[0])))

    cat_p = jnp.pad(category.astype(jnp.int32), (0, NP - N_NODES))
    rot2 = jnp.pad(rotation_z.astype(jnp.int32), (0, NP - N_NODES)).reshape(1, NP)
    dlt = jnp.concatenate(
        [dimension.T.astype(f32), location.T.astype(f32),
         jnp.ones((1, N_NODES), f32), jnp.zeros((1, N_NODES), f32)], axis=0)
    dlt = jnp.pad(dlt, ((0, 0), (0, NP - N_NODES)))
    Wc_p = W_cat
    Wn1 = W_node[:, 0:64]
    Wn2 = W_node[:, 64:96]
    Wn3 = W_node[:, 96:112]
    Wn4 = W_node[:, 112:128]
    bs2 = b_size.reshape(1, 16)
    bt2 = b_trans.reshape(1, 16)
    bn2 = b_node.reshape(1, 64)

    m_edge_t = pl.pallas_call(
        _edge_body,
        grid=(N_EDGES // E_BLK,),
        in_specs=[
            pl.BlockSpec((1, 1, E_BLK), lambda i: (i, 0, 0)),
            pl.BlockSpec((64, ED_P), lambda i: (0, 0)),
        ],
        out_specs=pl.BlockSpec((64, E_BLK), lambda i: (0, i)),
        out_shape=jax.ShapeDtypeStruct((64, N_EDGES), f32),
    )(eidx, WeT_p)

    sc_gather = pl.kernel(
        _sc_gather_body,
        out_type=jax.ShapeDtypeStruct((NP, 64), f32),
        mesh=plsc.VectorSubcoreMesh(core_axis_name="c", subcore_axis_name="s"),
        scratch_types=(
            [pltpu.VMEM((BPW,), jnp.int32)]
            + [pltpu.VMEM((CH, 64), f32)] * 2
            + [pltpu.SemaphoreType.DMA] * 4
        ),
        compiler_params=pltpu.CompilerParams(use_tc_tiling_on_sc=False),
    )
    gcat = sc_gather(cat_p, Wc_p)

    full = lambda shape: pl.BlockSpec(shape, lambda i: tuple(0 for _ in shape))
    m_node_t = pl.pallas_call(
        _node_body,
        grid=(NP // N_BLK,),
        in_specs=[
            pl.BlockSpec((N_BLK, 64), lambda i: (i, 0)),
            pl.BlockSpec((1, N_BLK), lambda i: (0, i)),
            pl.BlockSpec((8, N_BLK), lambda i: (0, i)),
            full((N_OR, 32)),
            full((64, 64)), full((64, 32)), full((64, 16)), full((64, 16)),
            full((16, 3)), full((16, 3)),
            full((1, 16)), full((1, 16)), full((1, 64)),
        ],
        out_specs=pl.BlockSpec((64, N_BLK), lambda i: (0, i)),
        out_shape=jax.ShapeDtypeStruct((64, NP), f32),
    )(gcat, rot2, dlt, W_orient, Wn1, Wn2, Wn3, Wn4,
      W_size, W_trans, bs2, bt2, bn2)

    return (m_node_t[:, :N_NODES].T, m_edge_t.T)
